# baseline (speedup 1.0000x reference)
import math

import jax
import jax.numpy as jnp
from jax import lax
from jax.experimental import pallas as pl
from jax.experimental.pallas import tpu as pltpu

N_SAMPLE = 20
EMB = 512
HEADS = 4
DH = EMB // HEADS
BATCH = 8
NPTS = 16384
TTOK = 64
NVIEW = 4
KPV = N_SAMPLE // NVIEW
LTOT = N_SAMPLE + TTOK


def _fused_body(xyz_ref, mask_ref, pf_ref, t_ref, wq_ref, bq_ref, wk_ref,
                bk_ref, wv_ref, bv_ref, wo_ref, bo_ref, out_ref,
                x_scr, blk_scr, d2a, d2b, d2c, d2d, sem):
    d2v_scr = (d2a, d2b, d2c, d2d)
    b = pl.program_id(0)
    x3 = xyz_ref[0]
    m = mask_ref[0]
    cnt = jnp.clip(jnp.sum(m, axis=1), 1.0, None)
    dist2 = jnp.zeros((NVIEW, NPTS), jnp.float32)
    for d in range(3):
        xd = x3[d:d + 1, :]
        cd = jnp.sum(m * xd, axis=1) / cnt
        t = xd - cd[:, None]
        dist2 = dist2 + t * t
    NB = NPTS // 128
    lane1 = lax.broadcasted_iota(jnp.int32, (1, 128), 1)
    inf = jnp.float32(jnp.inf)

    def lex_min(V, F):
        for sh in (64, 32, 16, 8, 4, 2, 1):
            V2 = pltpu.roll(V, sh, axis=1)
            F2 = pltpu.roll(F, sh, axis=1)
            better = (V2 < V) | ((V2 == V) & (F2 < F))
            V = jnp.where(better, V2, V)
            F = jnp.where(better, F2, F)
        return V, F

    idxs = []
    copies = []
    for v in range(NVIEW):
        dv2 = dist2[v:v + 1, :].reshape(NB, 128)
        d2v_scr[v][...] = dv2
        Ml = jnp.min(dv2, axis=1).reshape(1, NB)
        blk_vecs = []
        for k in range(KPV):
            _, Bm = lex_min(Ml, lane1)
            blk_vecs.append(Bm)
            Ml = jnp.where(lane1 == Bm, inf, Ml)
        cands = []
        for Bm in blk_vecs:
            blk = jnp.min(Bm)
            bd = d2v_scr[v][pl.ds(blk, 1), :]
            cands.append((bd, Bm * 128 + lane1))
        for k in range(KPV):
            V, F = cands[0]
            for V2, F2 in cands[1:]:
                better = (V2 < V) | ((V2 == V) & (F2 < F))
                V = jnp.where(better, V2, V)
                F = jnp.where(better, F2, F)
            _, gFv = lex_min(V, F)
            gF = jnp.min(gFv)
            r = v * KPV + k
            cp = pltpu.make_async_copy(
                pf_ref.at[b, :, pl.ds(pl.multiple_of((gF // 128) * 128, 128), 128)],
                blk_scr.at[:, pl.ds(r * 128, 128)],
                sem,
            )
            cp.start()
            idxs.append(gF)
            copies.append(cp)
            cands = [(jnp.where(Fj == gFv, inf, Vj), Fj) for Vj, Fj in cands]
    for cp in copies:
        cp.wait()
    lane128 = lax.broadcasted_iota(jnp.int32, (EMB, 128), 1)
    terms = []
    for r, mi in enumerate(idxs):
        blk = blk_scr[:, r * 128:(r + 1) * 128]
        rolled = pltpu.roll(blk, jnp.remainder(r - mi % 128, 128), axis=1)
        terms.append(jnp.where(lane128 == r, rolled, 0.0))
    while len(terms) > 1:
        terms = [terms[i] + terms[i + 1] if i + 1 < len(terms) else terms[i]
                 for i in range(0, len(terms), 2)]
    S = terms[0]
    eye = (lax.broadcasted_iota(jnp.int32, (EMB, EMB), 0)
           == lax.broadcasted_iota(jnp.int32, (EMB, EMB), 1)).astype(jnp.float32)
    St = lax.dot_general(S, eye, (((0,), (0,)), ((), ())),
                         preferred_element_type=jnp.float32)
    x_scr[0:N_SAMPLE, :] = St[0:N_SAMPLE, :]
    x_scr[N_SAMPLE:LTOT, :] = t_ref[0]
    x = x_scr[...]
    q = jnp.dot(x, wq_ref[...], preferred_element_type=jnp.float32) + bq_ref[...]
    k = jnp.dot(x, wk_ref[...], preferred_element_type=jnp.float32) + bk_ref[...]
    v = jnp.dot(x, wv_ref[...], preferred_element_type=jnp.float32) + bv_ref[...]
    scale = 1.0 / math.sqrt(DH)
    o_heads = []
    for h in range(HEADS):
        c0 = h * DH
        qh = q[:, c0:c0 + DH]
        kh = k[:, c0:c0 + DH]
        vh = v[:, c0:c0 + DH]
        s = lax.dot_general(qh, kh, (((1,), (1,)), ((), ())),
                            preferred_element_type=jnp.float32) * scale
        mx = jnp.max(s, axis=1, keepdims=True)
        e = jnp.exp(s - mx)
        a = e / jnp.sum(e, axis=1, keepdims=True)
        o_heads.append(jnp.dot(a, vh, preferred_element_type=jnp.float32))
    o = jnp.concatenate(o_heads, axis=1)
    out = jnp.dot(o, wo_ref[...], preferred_element_type=jnp.float32) + bo_ref[...]
    out_ref[0] = out


def _fused(xyz, masks, pf, t_feat, Wq, bq, Wk, bk, Wv, bv, Wo, bo,
           *, interpret=False):
    return pl.pallas_call(
        _fused_body,
        grid=(BATCH,),
        in_specs=[
            pl.BlockSpec((1, 3, NPTS), lambda b: (b, 0, 0)),
            pl.BlockSpec((1, NVIEW, NPTS), lambda b: (b, 0, 0)),
            pl.BlockSpec(memory_space=pl.ANY),
            pl.BlockSpec((1, TTOK, EMB), lambda b: (b, 0, 0)),
            pl.BlockSpec((EMB, EMB), lambda b: (0, 0)),
            pl.BlockSpec((1, EMB), lambda b: (0, 0)),
            pl.BlockSpec((EMB, EMB), lambda b: (0, 0)),
            pl.BlockSpec((1, EMB), lambda b: (0, 0)),
            pl.BlockSpec((EMB, EMB), lambda b: (0, 0)),
            pl.BlockSpec((1, EMB), lambda b: (0, 0)),
            pl.BlockSpec((EMB, EMB), lambda b: (0, 0)),
            pl.BlockSpec((1, EMB), lambda b: (0, 0)),
        ],
        out_specs=pl.BlockSpec((1, LTOT, EMB), lambda b: (b, 0, 0)),
        out_shape=jax.ShapeDtypeStruct((BATCH, LTOT, EMB), jnp.float32),
        scratch_shapes=[
            pltpu.VMEM((LTOT, EMB), jnp.float32),
            pltpu.VMEM((EMB, N_SAMPLE * 128), jnp.float32),
            pltpu.VMEM((NPTS // 128, 128), jnp.float32),
            pltpu.VMEM((NPTS // 128, 128), jnp.float32),
            pltpu.VMEM((NPTS // 128, 128), jnp.float32),
            pltpu.VMEM((NPTS // 128, 128), jnp.float32),
            pltpu.SemaphoreType.DMA,
        ],
        interpret=interpret,
    )(xyz, masks, pf, t_feat, Wq, bq, Wk, bk, Wv, bv, Wo, bo)


def kernel(point_features, point_masks, t_feat, t_mask, xyz, Wq, bq, Wk, bk,
           Wv, bv, Wo, bo):
    output = _fused(xyz, point_masks, point_features, t_feat,
                    Wq, bq.reshape(1, EMB), Wk, bk.reshape(1, EMB),
                    Wv, bv.reshape(1, EMB), Wo, bo.reshape(1, EMB))
    combined_mask = jnp.concatenate(
        [jnp.ones((BATCH, N_SAMPLE), dtype=bool), t_mask], axis=1)
    return (output, combined_mask)


# speedup vs baseline: 1.8959x; 1.8959x over previous
import math

import jax
import jax.numpy as jnp
from jax import lax
from jax.experimental import pallas as pl
from jax.experimental.pallas import tpu as pltpu

N_SAMPLE = 20
EMB = 512
HEADS = 4
DH = EMB // HEADS
BATCH = 8
NPTS = 16384
TTOK = 64
NVIEW = 4
KPV = N_SAMPLE // NVIEW
LTOT = N_SAMPLE + TTOK


def _fused_body(xyz_ref, mask_ref, pf_ref, t_ref, wq_ref, bq_ref, wk_ref,
                bk_ref, wv_ref, bv_ref, wo_ref, bo_ref, out_ref,
                x_scr, blk_scr, sem):
    b = pl.program_id(0)
    x3 = xyz_ref[0]
    m = mask_ref[0]
    cnt = jnp.clip(jnp.sum(m, axis=1), 1.0, None)
    dist2 = jnp.zeros((NVIEW, NPTS), jnp.float32)
    for d in range(3):
        xd = x3[d:d + 1, :]
        cd = jnp.sum(m * xd, axis=1) / cnt
        t = xd - cd[:, None]
        dist2 = dist2 + t * t
    lane = lax.broadcasted_iota(jnp.int32, (NVIEW, NPTS), 1)
    col16 = lax.broadcasted_iota(jnp.int32, (NVIEW, 16), 1)
    row16 = lax.broadcasted_iota(jnp.int32, (NVIEW, 16), 0)
    inf = jnp.float32(jnp.inf)
    arr = jnp.zeros((NVIEW, 16), jnp.int32)
    d2 = dist2
    for k in range(KPV):
        mn = jnp.min(d2, axis=1, keepdims=True)
        cand = jnp.where(d2 == mn, lane, NPTS)
        mi = jnp.min(cand, axis=1, keepdims=True)
        arr = jnp.where(col16 == k, mi, arr)
        d2 = jnp.where(lane == mi, inf, d2)
    idxs = []
    copies = []
    for v in range(NVIEW):
        for k in range(KPV):
            gF = jnp.sum(jnp.where((row16 == v) & (col16 == k), arr, 0))
            r = v * KPV + k
            cp = pltpu.make_async_copy(
                pf_ref.at[b, :, pl.ds(pl.multiple_of((gF // 128) * 128, 128), 128)],
                blk_scr.at[:, pl.ds(r * 128, 128)],
                sem,
            )
            cp.start()
            idxs.append(gF)
            copies.append(cp)
    for cp in copies:
        cp.wait()
    lane128 = lax.broadcasted_iota(jnp.int32, (EMB, 128), 1)
    terms = []
    for r, mi in enumerate(idxs):
        blk = blk_scr[:, r * 128:(r + 1) * 128]
        rolled = pltpu.roll(blk, jnp.remainder(r - mi % 128, 128), axis=1)
        terms.append(jnp.where(lane128 == r, rolled, 0.0))
    while len(terms) > 1:
        terms = [terms[i] + terms[i + 1] if i + 1 < len(terms) else terms[i]
                 for i in range(0, len(terms), 2)]
    S = terms[0]
    eye = (lax.broadcasted_iota(jnp.int32, (EMB, EMB), 0)
           == lax.broadcasted_iota(jnp.int32, (EMB, EMB), 1)).astype(jnp.float32)
    St = lax.dot_general(S, eye, (((0,), (0,)), ((), ())),
                         preferred_element_type=jnp.float32)
    x_scr[0:N_SAMPLE, :] = St[0:N_SAMPLE, :]
    x_scr[N_SAMPLE:LTOT, :] = t_ref[0]
    x = x_scr[...]
    q = jnp.dot(x, wq_ref[...], preferred_element_type=jnp.float32) + bq_ref[...]
    k = jnp.dot(x, wk_ref[...], preferred_element_type=jnp.float32) + bk_ref[...]
    v = jnp.dot(x, wv_ref[...], preferred_element_type=jnp.float32) + bv_ref[...]
    scale = 1.0 / math.sqrt(DH)
    o_heads = []
    for h in range(HEADS):
        c0 = h * DH
        qh = q[:, c0:c0 + DH]
        kh = k[:, c0:c0 + DH]
        vh = v[:, c0:c0 + DH]
        s = lax.dot_general(qh, kh, (((1,), (1,)), ((), ())),
                            preferred_element_type=jnp.float32) * scale
        mx = jnp.max(s, axis=1, keepdims=True)
        e = jnp.exp(s - mx)
        a = e / jnp.sum(e, axis=1, keepdims=True)
        o_heads.append(jnp.dot(a, vh, preferred_element_type=jnp.float32))
    o = jnp.concatenate(o_heads, axis=1)
    out = jnp.dot(o, wo_ref[...], preferred_element_type=jnp.float32) + bo_ref[...]
    out_ref[0] = out


def _fused(xyz, masks, pf, t_feat, Wq, bq, Wk, bk, Wv, bv, Wo, bo,
           *, interpret=False):
    return pl.pallas_call(
        _fused_body,
        grid=(BATCH,),
        in_specs=[
            pl.BlockSpec((1, 3, NPTS), lambda b: (b, 0, 0)),
            pl.BlockSpec((1, NVIEW, NPTS), lambda b: (b, 0, 0)),
            pl.BlockSpec(memory_space=pl.ANY),
            pl.BlockSpec((1, TTOK, EMB), lambda b: (b, 0, 0)),
            pl.BlockSpec((EMB, EMB), lambda b: (0, 0)),
            pl.BlockSpec((1, EMB), lambda b: (0, 0)),
            pl.BlockSpec((EMB, EMB), lambda b: (0, 0)),
            pl.BlockSpec((1, EMB), lambda b: (0, 0)),
            pl.BlockSpec((EMB, EMB), lambda b: (0, 0)),
            pl.BlockSpec((1, EMB), lambda b: (0, 0)),
            pl.BlockSpec((EMB, EMB), lambda b: (0, 0)),
            pl.BlockSpec((1, EMB), lambda b: (0, 0)),
        ],
        out_specs=pl.BlockSpec((1, LTOT, EMB), lambda b: (b, 0, 0)),
        out_shape=jax.ShapeDtypeStruct((BATCH, LTOT, EMB), jnp.float32),
        scratch_shapes=[
            pltpu.VMEM((LTOT, EMB), jnp.float32),
            pltpu.VMEM((EMB, N_SAMPLE * 128), jnp.float32),
            pltpu.SemaphoreType.DMA,
        ],
        interpret=interpret,
    )(xyz, masks, pf, t_feat, Wq, bq, Wk, bk, Wv, bv, Wo, bo)


def kernel(point_features, point_masks, t_feat, t_mask, xyz, Wq, bq, Wk, bk,
           Wv, bv, Wo, bo):
    output = _fused(xyz, point_masks, point_features, t_feat,
                    Wq, bq.reshape(1, EMB), Wk, bk.reshape(1, EMB),
                    Wv, bv.reshape(1, EMB), Wo, bo.reshape(1, EMB))
    combined_mask = jnp.concatenate(
        [jnp.ones((BATCH, N_SAMPLE), dtype=bool), t_mask], axis=1)
    return (output, combined_mask)
